# baseline (speedup 1.0000x reference)
import functools

import jax
import jax.numpy as jnp
import numpy as np
from jax import lax
from jax.experimental import pallas as pl
from jax.experimental.pallas import tpu as pltpu
from jax.experimental.pallas import tpu_sc as plsc

_B = 16384
_R = 128
_C = 128

_NC = 2
_NS = 16
_L = 16
_NW = _NC * _NS
_ROWS_PER_TILE = _B // _NW

_KEY_HI = np.int32(0)
_KEY_LO = np.int32(42)


def _tf_rounds(x0, x1, rots):
    for r in rots:
        x0 = x0 + x1
        x1 = lax.shift_left(x1, np.int32(r)) | lax.shift_right_logical(
            x1, np.int32(32 - r)
        )
        x1 = x1 ^ x0
    return x0, x1


def _threefry2x32(x0, x1, k0, k1):
    ks2 = np.int32(k0 ^ k1 ^ np.int32(0x1BD11BDA))
    ks = (k0, k1, ks2)
    rot_a = (13, 15, 26, 6)
    rot_b = (17, 29, 16, 24)
    x0 = x0 + ks[0]
    x1 = x1 + ks[1]
    x0, x1 = _tf_rounds(x0, x1, rot_a)
    x0 = x0 + ks[1]
    x1 = x1 + np.int32(ks[2] + 1)
    x0, x1 = _tf_rounds(x0, x1, rot_b)
    x0 = x0 + ks[2]
    x1 = x1 + np.int32(ks[0] + 2)
    x0, x1 = _tf_rounds(x0, x1, rot_a)
    x0 = x0 + ks[0]
    x1 = x1 + np.int32(ks[1] + 3)
    x0, x1 = _tf_rounds(x0, x1, rot_b)
    x0 = x0 + ks[1]
    x1 = x1 + np.int32(ks[2] + 4)
    x0, x1 = _tf_rounds(x0, x1, rot_a)
    x0 = x0 + ks[2]
    x1 = x1 + np.int32(ks[0] + 5)
    return x0, x1


_sc_mesh = plsc.VectorSubcoreMesh(core_axis_name="c", subcore_axis_name="s")


@functools.partial(
    pl.kernel,
    mesh=_sc_mesh,
    out_type=[
        jax.ShapeDtypeStruct((_B,), jnp.int32),
        jax.ShapeDtypeStruct((_B,), jnp.int32),
    ],
    scratch_types=[
        pltpu.VMEM((_ROWS_PER_TILE,), jnp.int32),
        pltpu.VMEM((_ROWS_PER_TILE,), jnp.int32),
    ],
)
def _sc_bits(outa_hbm, outb_hbm, bufa, bufb):
    wid = lax.axis_index("s") * _NC + lax.axis_index("c")
    base = wid * _ROWS_PER_TILE
    lanes = lax.iota(jnp.int32, _L)

    def body(v, carry):
        row = base + v * _L
        cnt = (jnp.full((_L,), row * 2, jnp.int32) + lanes * np.int32(2))
        za = jnp.zeros((_L,), jnp.int32)
        a0, a1 = _threefry2x32(za, cnt, _KEY_HI, _KEY_LO)
        b0, b1 = _threefry2x32(za, cnt + np.int32(1), _KEY_HI, _KEY_LO)
        bufa[pl.ds(v * _L, _L)] = a0 ^ a1
        bufb[pl.ds(v * _L, _L)] = b0 ^ b1
        return carry

    lax.fori_loop(0, _ROWS_PER_TILE // _L, body, 0)
    pltpu.sync_copy(bufa, outa_hbm.at[pl.ds(base, _ROWS_PER_TILE)])
    pltpu.sync_copy(bufb, outb_hbm.at[pl.ds(base, _ROWS_PER_TILE)])


def _bits_to_gumbel(bits_i32):
    tiny = np.float32(np.finfo(np.float32).tiny)
    bits = lax.bitcast_convert_type(bits_i32, jnp.uint32)
    mant = (bits >> np.uint32(9)) | np.uint32(0x3F800000)
    fl = lax.bitcast_convert_type(mant, jnp.float32) - np.float32(1.0)
    u = jnp.maximum(tiny, fl * (np.float32(1.0) - tiny) + tiny)
    return -jnp.log(-jnp.log(u))


def _combine_body(p_ref, a_ref, b_ref, o_ref):
    p = p_ref[...]
    g0 = _bits_to_gumbel(a_ref[...])
    g1 = _bits_to_gumbel(b_ref[...])
    v0 = jnp.log(np.float32(1.0) - p) + g0
    v1 = jnp.log(p) + g1
    o_ref[...] = (v1 > v0).astype(jnp.float32)


def kernel(p_t):
    ba, bb = _sc_bits()
    p2 = p_t.reshape(_R, _C)
    out = pl.pallas_call(
        _combine_body,
        out_shape=jax.ShapeDtypeStruct((_R, _C), jnp.float32),
    )(p2, ba.reshape(_R, _C), bb.reshape(_R, _C))
    return out.reshape(_B, 1, 1)


# speedup vs baseline: 10.8156x; 10.8156x over previous
import jax
import jax.numpy as jnp
import numpy as np
from jax.experimental import pallas as pl

_B = 16384
_R = 128
_C = 128

_KEY_HI = np.uint32(0)
_KEY_LO = np.uint32(42)


def _threefry2x32(x0, x1, k0, k1):
    ks2 = k0 ^ k1 ^ np.uint32(0x1BD11BDA)
    ks = (k0, k1, ks2)
    rot_a = (13, 15, 26, 6)
    rot_b = (17, 29, 16, 24)

    def rounds(x0, x1, rots):
        for r in rots:
            x0 = x0 + x1
            x1 = (x1 << np.uint32(r)) | (x1 >> np.uint32(32 - r))
            x1 = x1 ^ x0
        return x0, x1

    x0 = x0 + ks[0]
    x1 = x1 + ks[1]
    x0, x1 = rounds(x0, x1, rot_a)
    x0 = x0 + ks[1]
    x1 = x1 + ks[2] + np.uint32(1)
    x0, x1 = rounds(x0, x1, rot_b)
    x0 = x0 + ks[2]
    x1 = x1 + ks[0] + np.uint32(2)
    x0, x1 = rounds(x0, x1, rot_a)
    x0 = x0 + ks[0]
    x1 = x1 + ks[1] + np.uint32(3)
    x0, x1 = rounds(x0, x1, rot_b)
    x0 = x0 + ks[1]
    x1 = x1 + ks[2] + np.uint32(4)
    x0, x1 = rounds(x0, x1, rot_a)
    x0 = x0 + ks[2]
    x1 = x1 + ks[0] + np.uint32(5)
    return x0, x1


def _bits_to_gumbel(bits):
    tiny = np.float32(np.finfo(np.float32).tiny)
    mant = (bits >> np.uint32(9)) | np.uint32(0x3F800000)
    fl = jax.lax.bitcast_convert_type(mant, jnp.float32) - np.float32(1.0)
    u = fl * (np.float32(1.0) - tiny) + tiny
    return -jnp.log(-jnp.log(u))


def _sample_body(p_ref, o_ref):
    p = p_ref[...]
    row = jax.lax.broadcasted_iota(jnp.uint32, (_R, _C), 0)
    col = jax.lax.broadcasted_iota(jnp.uint32, (_R, _C), 1)
    flat2 = (row * np.uint32(_C) + col) * np.uint32(2)

    a0, a1 = _threefry2x32(jnp.zeros_like(flat2), flat2, _KEY_HI, _KEY_LO)
    b0, b1 = _threefry2x32(
        jnp.zeros_like(flat2), flat2 + np.uint32(1), _KEY_HI, _KEY_LO
    )
    g0 = _bits_to_gumbel(a0 ^ a1)
    g1 = _bits_to_gumbel(b0 ^ b1)

    v0 = jnp.log(np.float32(1.0) - p) + g0
    v1 = jnp.log(p) + g1
    o_ref[...] = (v1 > v0).astype(jnp.float32)


def kernel(p_t):
    p2 = p_t.reshape(_R, _C)
    out = pl.pallas_call(
        _sample_body,
        out_shape=jax.ShapeDtypeStruct((_R, _C), jnp.float32),
    )(p2)
    return out.reshape(_B, 1, 1)


# speedup vs baseline: 10.8356x; 1.0018x over previous
import jax
import jax.numpy as jnp
import numpy as np
from jax.experimental import pallas as pl

_B = 16384
_R = 128
_C = 128

_KEY_HI = np.uint32(0)
_KEY_LO = np.uint32(42)


def _threefry2x32(x0, x1, k0, k1):
    ks2 = k0 ^ k1 ^ np.uint32(0x1BD11BDA)
    ks = (k0, k1, ks2)
    rot_a = (13, 15, 26, 6)
    rot_b = (17, 29, 16, 24)

    def rounds(x0, x1, rots):
        for r in rots:
            x0 = x0 + x1
            x1 = (x1 << np.uint32(r)) | (x1 >> np.uint32(32 - r))
            x1 = x1 ^ x0
        return x0, x1

    x0 = x0 + ks[0]
    x1 = x1 + ks[1]
    x0, x1 = rounds(x0, x1, rot_a)
    x0 = x0 + ks[1]
    x1 = x1 + ks[2] + np.uint32(1)
    x0, x1 = rounds(x0, x1, rot_b)
    x0 = x0 + ks[2]
    x1 = x1 + ks[0] + np.uint32(2)
    x0, x1 = rounds(x0, x1, rot_a)
    x0 = x0 + ks[0]
    x1 = x1 + ks[1] + np.uint32(3)
    x0, x1 = rounds(x0, x1, rot_b)
    x0 = x0 + ks[1]
    x1 = x1 + ks[2] + np.uint32(4)
    x0, x1 = rounds(x0, x1, rot_a)
    x0 = x0 + ks[2]
    x1 = x1 + ks[0] + np.uint32(5)
    return x0, x1


def _bits_to_gumbel(bits):
    tiny = np.float32(np.finfo(np.float32).tiny)
    mant = (bits >> np.uint32(9)) | np.uint32(0x3F800000)
    fl = jax.lax.bitcast_convert_type(mant, jnp.float32) - np.float32(1.0)
    u = fl * (np.float32(1.0) - tiny) + tiny
    return -jnp.log(-jnp.log(u))


def _sample_body(p_ref, o_ref):
    p = p_ref[...]
    row = jax.lax.broadcasted_iota(jnp.uint32, (_R, _C), 0)
    col = jax.lax.broadcasted_iota(jnp.uint32, (_R, _C), 1)
    flat2 = (row * np.uint32(_C) + col) * np.uint32(2)

    a0, a1 = _threefry2x32(jnp.zeros_like(flat2), flat2, _KEY_HI, _KEY_LO)
    b0, b1 = _threefry2x32(
        jnp.zeros_like(flat2), flat2 + np.uint32(1), _KEY_HI, _KEY_LO
    )
    g0 = _bits_to_gumbel(a0 ^ a1)
    g1 = _bits_to_gumbel(b0 ^ b1)

    v0 = jnp.log(np.float32(1.0) - p) + g0
    v1 = jnp.log(p) + g1
    o_ref[...] = (v1 > v0).astype(jnp.float32)


def kernel(p_t):
    p2 = p_t.reshape(_R, _C)
    from jax.experimental.pallas import tpu as pltpu

    out = pl.pallas_call(
        _sample_body,
        out_shape=jax.ShapeDtypeStruct((_R, _C), jnp.float32),
        compiler_params=pltpu.CompilerParams(skip_device_barrier=True),
    )(p2)
    return out.reshape(_B, 1, 1)


# speedup vs baseline: 10.8420x; 1.0006x over previous
import jax
import jax.numpy as jnp
import numpy as np
from jax.experimental import pallas as pl

_B = 16384
_R = 128
_C = 128

_KEY_HI = np.uint32(0)
_KEY_LO = np.uint32(42)


def _threefry2x32(x0, x1, k0, k1):
    ks2 = k0 ^ k1 ^ np.uint32(0x1BD11BDA)
    ks = (k0, k1, ks2)
    rot_a = (13, 15, 26, 6)
    rot_b = (17, 29, 16, 24)

    def rounds(x0, x1, rots):
        for r in rots:
            x0 = x0 + x1
            x1 = (x1 << np.uint32(r)) | (x1 >> np.uint32(32 - r))
            x1 = x1 ^ x0
        return x0, x1

    x0 = x0 + ks[0]
    x1 = x1 + ks[1]
    x0, x1 = rounds(x0, x1, rot_a)
    x0 = x0 + ks[1]
    x1 = x1 + ks[2] + np.uint32(1)
    x0, x1 = rounds(x0, x1, rot_b)
    x0 = x0 + ks[2]
    x1 = x1 + ks[0] + np.uint32(2)
    x0, x1 = rounds(x0, x1, rot_a)
    x0 = x0 + ks[0]
    x1 = x1 + ks[1] + np.uint32(3)
    x0, x1 = rounds(x0, x1, rot_b)
    x0 = x0 + ks[1]
    x1 = x1 + ks[2] + np.uint32(4)
    x0, x1 = rounds(x0, x1, rot_a)
    x0 = x0 + ks[2]
    x1 = x1 + ks[0] + np.uint32(5)
    return x0, x1


def _bits_to_gumbel(bits):
    tiny = np.float32(np.finfo(np.float32).tiny)
    mant = (bits >> np.uint32(9)) | np.uint32(0x3F800000)
    fl = jax.lax.bitcast_convert_type(mant, jnp.float32) - np.float32(1.0)
    u = fl * (np.float32(1.0) - tiny) + tiny
    return -jnp.log(-jnp.log(u))


def _sample_body(p_ref, o_ref):
    p = p_ref[...]
    row = jax.lax.broadcasted_iota(jnp.uint32, (_R, _C), 0)
    col = jax.lax.broadcasted_iota(jnp.uint32, (_R, _C), 1)
    flat2 = (row * np.uint32(_C) + col) * np.uint32(2)

    a0, a1 = _threefry2x32(jnp.zeros_like(flat2), flat2, _KEY_HI, _KEY_LO)
    b0, b1 = _threefry2x32(
        jnp.zeros_like(flat2), flat2 + np.uint32(1), _KEY_HI, _KEY_LO
    )
    g0 = _bits_to_gumbel(a0 ^ a1)
    g1 = _bits_to_gumbel(b0 ^ b1)

    v0 = jnp.log(np.float32(1.0) - p) + g0
    v1 = jnp.log(p) + g1
    o_ref[...] = (v1 > v0).astype(jnp.float32)


def kernel(p_t):
    p2 = p_t.reshape(_R, _C)
    out = pl.pallas_call(
        _sample_body,
        out_shape=jax.ShapeDtypeStruct((_R, _C), jnp.float32),
    )(p2)
    return out.reshape(_B, 1, 1)
